# speedup vs baseline: 1.9862x; 1.9862x over previous
import jax
import jax.numpy as jnp
from jax import lax
from jax.experimental import pallas as pl
from jax.experimental.pallas import tpu as pltpu
from jax.experimental.pallas import tpu_sc as plsc

D_MODEL = 1024
NC = 2
NS = 16
NW = NC * NS

K = 32


def _gather_body(x_hbm, pe_hbm, out_hbm, idx_v, rows_v, sem):
    c = lax.axis_index("c")
    s = lax.axis_index("s")
    wid = s * NC + c
    n_chunks = idx_v.shape[0]
    n_per_w = n_chunks * idx_v.shape[1]
    pltpu.sync_copy(x_hbm.at[wid], idx_v)

    @pl.loop(0, n_chunks)
    def _(j):
        pltpu.async_copy(pe_hbm.at[idx_v.at[j]], rows_v, sem).wait()
        pltpu.sync_copy(rows_v, out_hbm.at[pl.ds(wid * n_per_w + j * K, K)])


def kernel(x, pe):
    b, l = x.shape
    total = b * l
    n_per_w = total // NW
    n_chunks = n_per_w // K
    x_resh = x.reshape(NW, n_chunks, K).astype(jnp.int32)

    mesh = plsc.VectorSubcoreMesh(core_axis_name="c", subcore_axis_name="s")
    out = pl.kernel(
        _gather_body,
        out_type=jax.ShapeDtypeStruct((total, D_MODEL), jnp.float32),
        mesh=mesh,
        scratch_types=[
            pltpu.VMEM((n_chunks, K), jnp.int32),
            pltpu.VMEM((K, D_MODEL), jnp.float32),
            pltpu.SemaphoreType.DMA,
        ],
    )(x_resh, pe)
    return out.reshape(b, l, D_MODEL)


# speedup vs baseline: 2.3837x; 1.2001x over previous
import jax
import jax.numpy as jnp
from jax import lax
from jax.experimental import pallas as pl
from jax.experimental.pallas import tpu as pltpu
from jax.experimental.pallas import tpu_sc as plsc

D_MODEL = 1024
NC = 2
NS = 16
NW = NC * NS

K = 32


def _gather_body(x_hbm, pe_hbm, out_hbm, idx_v, rows_a, rows_b, sem_a, sem_b):
    c = lax.axis_index("c")
    s = lax.axis_index("s")
    wid = s * NC + c
    n_chunks = idx_v.shape[0]
    n_per_w = n_chunks * idx_v.shape[1]
    bufs = (rows_a, rows_b)
    sems = (sem_a, sem_b)
    pltpu.sync_copy(x_hbm.at[wid], idx_v)
    pltpu.async_copy(pe_hbm.at[idx_v.at[0]], rows_a, sem_a)

    @pl.loop(0, n_chunks, step=2)
    def _(j):
        for p in range(2):
            jj = j + p

            @pl.when(jj + 1 < n_chunks)
            def _():
                pltpu.async_copy(
                    pe_hbm.at[idx_v.at[jj + 1]], bufs[1 - p], sems[1 - p])

            pltpu.make_async_copy(
                pe_hbm.at[idx_v.at[jj]], bufs[p], sems[p]).wait()
            pltpu.sync_copy(
                bufs[p], out_hbm.at[pl.ds(wid * n_per_w + jj * K, K)])


def kernel(x, pe):
    b, l = x.shape
    total = b * l
    n_per_w = total // NW
    n_chunks = n_per_w // K
    x_resh = x.reshape(NW, n_chunks, K).astype(jnp.int32)

    mesh = plsc.VectorSubcoreMesh(core_axis_name="c", subcore_axis_name="s")
    out = pl.kernel(
        _gather_body,
        out_type=jax.ShapeDtypeStruct((total, D_MODEL), jnp.float32),
        mesh=mesh,
        scratch_types=[
            pltpu.VMEM((n_chunks, K), jnp.int32),
            pltpu.VMEM((K, D_MODEL), jnp.float32),
            pltpu.VMEM((K, D_MODEL), jnp.float32),
            pltpu.SemaphoreType.DMA,
            pltpu.SemaphoreType.DMA,
        ],
    )(x_resh, pe)
    return out.reshape(b, l, D_MODEL)
